# baseline (speedup 1.0000x reference)
import functools

import jax
import jax.numpy as jnp
from jax import lax
from jax.experimental import pallas as pl
from jax.experimental.pallas import tpu as pltpu
from jax.experimental.pallas import tpu_sc as plsc

N = 10000
E = 160000
FEAT = 256
HID = 512
TDIM = 128
H = HID + TDIM

NB = 1000
EB = 2000
PW = H // 2
NCHUNK = 4
CW = H // NCHUNK

NUM_TILES = 32
EPT = E // NUM_TILES
GK = 40
EPS = E // 16
SK = 80
NPS = 1000

_PREC = lax.Precision.HIGHEST


def _dotT(x, w):
    return lax.dot_general(x, w, (((1,), (1,)), ((), ())), precision=_PREC)


def _ln(v, g, b, eps=1e-5):
    m = jnp.mean(v, axis=-1, keepdims=True)
    var = jnp.mean((v - m) ** 2, axis=-1, keepdims=True)
    return (v - m) * lax.rsqrt(var + eps) * g + b


def _encode_body(nf_ref, ts_ref, ew_ref, eb_ref, eg_ref, ebeta_ref,
                 tw_ref, tb_ref, tg_ref, tbeta_ref, out_ref):
    h = _dotT(nf_ref[...], ew_ref[...]) + eb_ref[...]
    h = jax.nn.relu(_ln(h, eg_ref[...], ebeta_ref[...]))
    t = ts_ref[...] * tw_ref[...] + tb_ref[...]
    t = jax.nn.relu(_ln(t, tg_ref[...], tbeta_ref[...]))
    out_ref[:, :HID] = h
    out_ref[:, HID:] = t


def _encode(nf, ts, ew, eb, eg, ebeta, tw, tb, tg, tbeta):
    row = lambda d: pl.BlockSpec((1, d), lambda i: (0, 0))
    return pl.pallas_call(
        _encode_body,
        grid=(N // NB,),
        in_specs=[
            pl.BlockSpec((NB, FEAT), lambda i: (i, 0)),
            pl.BlockSpec((NB, 1), lambda i: (i, 0)),
            pl.BlockSpec((HID, FEAT), lambda i: (0, 0)),
            row(HID), row(HID), row(HID),
            row(TDIM), row(TDIM), row(TDIM), row(TDIM),
        ],
        out_specs=pl.BlockSpec((NB, H), lambda i: (i, 0)),
        out_shape=jax.ShapeDtypeStruct((N, H), jnp.float32),
    )(nf, ts, ew, eb, eg, ebeta, tw, tb, tg, tbeta)


def _pack2(v):
    lo = lax.bitcast_convert_type(v[:, :PW].astype(jnp.bfloat16), jnp.uint16)
    hi = lax.bitcast_convert_type(v[:, PW:].astype(jnp.bfloat16), jnp.uint16)
    packed = lo.astype(jnp.uint32) | (hi.astype(jnp.uint32) << 16)
    return lax.bitcast_convert_type(packed, jnp.float32)


def _unpack2(v):
    u = lax.bitcast_convert_type(v, jnp.uint32)
    lo = lax.bitcast_convert_type((u & 0xFFFF).astype(jnp.uint16), jnp.bfloat16)
    hi = lax.bitcast_convert_type((u >> 16).astype(jnp.uint16), jnp.bfloat16)
    return jnp.concatenate([lo.astype(jnp.float32), hi.astype(jnp.float32)],
                           axis=-1)


def _ab_body(x_ref, ws_ref, wd_ref, mb_ref, a_ref, b_ref):
    x = x_ref[...]
    a_ref[...] = _pack2(_dotT(x, ws_ref[...]))
    b_ref[...] = _pack2(_dotT(x, wd_ref[...]) + mb_ref[...])


def _ab(x, ws, wd, mb):
    return pl.pallas_call(
        _ab_body,
        grid=(N // NB,),
        in_specs=[
            pl.BlockSpec((NB, H), lambda i: (i, 0)),
            pl.BlockSpec((H, H), lambda i: (0, 0)),
            pl.BlockSpec((H, H), lambda i: (0, 0)),
            pl.BlockSpec((1, H), lambda i: (0, 0)),
        ],
        out_specs=[pl.BlockSpec((NB, PW), lambda i: (i, 0)),
                   pl.BlockSpec((NB, PW), lambda i: (i, 0))],
        out_shape=[jax.ShapeDtypeStruct((N, PW), jnp.float32),
                   jax.ShapeDtypeStruct((N, PW), jnp.float32)],
    )(x, ws, wd, mb)


def _edge_math_body(g1_ref, g2_ref, w_ref, mg_ref, mbeta_ref, out_ref):
    s = _unpack2(g1_ref[...]) + _unpack2(g2_ref[...])
    y = jax.nn.relu(_ln(s, mg_ref[...], mbeta_ref[...])) * w_ref[...]
    for c in range(NCHUNK):
        out_ref[c] = y[:, c * CW:(c + 1) * CW]


def _edge_math(g1, g2, w, mg, mbeta):
    return pl.pallas_call(
        _edge_math_body,
        grid=(E // EB,),
        in_specs=[
            pl.BlockSpec((EB, PW), lambda i: (i, 0)),
            pl.BlockSpec((EB, PW), lambda i: (i, 0)),
            pl.BlockSpec((EB, 1), lambda i: (i, 0)),
            pl.BlockSpec((1, H), lambda i: (0, 0)),
            pl.BlockSpec((1, H), lambda i: (0, 0)),
        ],
        out_specs=pl.BlockSpec((NCHUNK, EB, CW), lambda i: (0, i, 0)),
        out_shape=jax.ShapeDtypeStruct((NCHUNK, E, CW), jnp.float32),
    )(g1, g2, w, mg, mbeta)


def _update_body(x_ref, ms_ref, cnt_ref, uw_ref, ub_ref, ug_ref, ubeta_ref,
                 gw_ref, gb_ref, out_ref):
    x = x_ref[...]
    msum = jnp.concatenate([ms_ref[c] for c in range(NCHUNK)], axis=-1)
    cnt = cnt_ref[0, :, 0:1] + cnt_ref[1, :, 0:1]
    valid = (cnt > 0).astype(jnp.float32)
    messages = msum / (cnt + 1e-8) * valid
    tw = jax.nn.sigmoid(jnp.sum(x * gw_ref[...], axis=-1, keepdims=True)
                        + gb_ref[...])
    combined = jnp.concatenate([x, messages], axis=-1)
    h_new = _dotT(combined, uw_ref[...]) + ub_ref[...]
    h_new = jax.nn.relu(_ln(h_new, ug_ref[...], ubeta_ref[...]))
    out_ref[...] = tw * h_new + (1.0 - tw) * x


def _update(x, msum, cnts, uw, ub, ug, ubeta, gw, gb):
    return pl.pallas_call(
        _update_body,
        grid=(N // NB,),
        in_specs=[
            pl.BlockSpec((NB, H), lambda i: (i, 0)),
            pl.BlockSpec((NCHUNK, NB, CW), lambda i: (0, i, 0)),
            pl.BlockSpec((2, NB, 16), lambda i: (0, i, 0)),
            pl.BlockSpec((H, 2 * H), lambda i: (0, 0)),
            pl.BlockSpec((1, H), lambda i: (0, 0)),
            pl.BlockSpec((1, H), lambda i: (0, 0)),
            pl.BlockSpec((1, H), lambda i: (0, 0)),
            pl.BlockSpec((1, H), lambda i: (0, 0)),
            pl.BlockSpec((1, 1), lambda i: (0, 0)),
        ],
        out_specs=pl.BlockSpec((NB, H), lambda i: (i, 0)),
        out_shape=jax.ShapeDtypeStruct((N, H), jnp.float32),
    )(x, msum, cnts, uw, ub, ug, ubeta, gw, gb)


def _out_body(x_ref, ow_ref, ob_ref, out_ref):
    o = _dotT(x_ref[...], ow_ref[...]) + ob_ref[...]
    nrm = jnp.sqrt(jnp.sum(o * o, axis=-1, keepdims=True))
    out_ref[...] = o / jnp.maximum(nrm, 1e-12)


def _out_proj(x, ow, ob):
    return pl.pallas_call(
        _out_body,
        grid=(N // NB,),
        in_specs=[
            pl.BlockSpec((NB, H), lambda i: (i, 0)),
            pl.BlockSpec((FEAT, H), lambda i: (0, 0)),
            pl.BlockSpec((1, FEAT), lambda i: (0, 0)),
        ],
        out_specs=pl.BlockSpec((NB, FEAT), lambda i: (i, 0)),
        out_shape=jax.ShapeDtypeStruct((N, FEAT), jnp.float32),
    )(x, ow, ob)


@functools.cache
def _sc_mesh():
    return plsc.VectorSubcoreMesh(core_axis_name="c", subcore_axis_name="s")


@functools.cache
def _sc_gather_kernel():
    return pl.kernel(
        _sc_gather_body,
        out_type=[jax.ShapeDtypeStruct((E, PW), jnp.float32),
                  jax.ShapeDtypeStruct((E, PW), jnp.float32)],
        mesh=_sc_mesh(),
        scratch_types=[
            pltpu.VMEM((EPT,), jnp.int32),
            pltpu.VMEM((EPT,), jnp.int32),
            pltpu.VMEM((GK, PW), jnp.float32),
            pltpu.VMEM((GK, PW), jnp.float32),
            pltpu.SemaphoreType.DMA,
        ],
        compiler_params=pltpu.CompilerParams(use_tc_tiling_on_sc=False),
    )


def _sc_gather_body(a_hbm, b_hbm, src_hbm, dst_hbm, g1_hbm, g2_hbm,
                    idx_s, idx_d, rows_a, rows_b, sem):
    c = lax.axis_index("c")
    s = lax.axis_index("s")
    wid = s * 2 + c
    base = wid * EPT
    pltpu.sync_copy(src_hbm.at[pl.ds(base, EPT)], idx_s)
    pltpu.sync_copy(dst_hbm.at[pl.ds(base, EPT)], idx_d)

    @pl.loop(0, EPT, step=GK)
    def _chunk(off):
        ca = pltpu.async_copy(a_hbm.at[idx_s.at[pl.ds(off, GK)]], rows_a, sem)
        cb = pltpu.async_copy(b_hbm.at[idx_d.at[pl.ds(off, GK)]], rows_b, sem)
        ca.wait()
        cb.wait()
        pltpu.sync_copy(rows_a, g1_hbm.at[pl.ds(base + off, GK)])
        pltpu.sync_copy(rows_b, g2_hbm.at[pl.ds(base + off, GK)])


@functools.cache
def _sc_scatter_kernel():
    return pl.kernel(
        _sc_scatter_body,
        out_type=jax.ShapeDtypeStruct((NCHUNK, N, CW), jnp.float32),
        mesh=_sc_mesh(),
        scratch_types=[
            pltpu.VMEM((SK, CW), jnp.float32),
            pltpu.VMEM((SK,), jnp.int32),
            pltpu.VMEM_SHARED((N, CW), jnp.float32),
        ],
        compiler_params=pltpu.CompilerParams(use_tc_tiling_on_sc=False),
    )


def _sc_scatter_body(em_hbm, dst_hbm, zeros_hbm, msum_hbm, rows_v, idx_v, tbl):
    c = lax.axis_index("c")
    s = lax.axis_index("s")
    base = s * EPS
    for cp in range(NCHUNK // 2):
        chunk = c * (NCHUNK // 2) + cp

        @pl.when(s == 0)
        def _zero():
            pltpu.sync_copy(zeros_hbm, tbl)

        plsc.subcore_barrier()

        @pl.loop(0, EPS, step=SK)
        def _chunk_loop(off):
            pltpu.sync_copy(dst_hbm.at[pl.ds(base + off, SK)], idx_v)
            pltpu.sync_copy(em_hbm.at[chunk, pl.ds(base + off, SK)], rows_v)
            pltpu.sync_copy(rows_v, tbl.at[idx_v], add=True)

        plsc.subcore_barrier()

        @pl.when(s < N // NPS)
        def _writeback():
            pltpu.sync_copy(tbl.at[pl.ds(s * NPS, NPS)],
                            msum_hbm.at[chunk, pl.ds(s * NPS, NPS)])

        plsc.subcore_barrier()


@functools.cache
def _sc_counts_kernel():
    return pl.kernel(
        _sc_counts_body,
        out_type=jax.ShapeDtypeStruct((2, N, 16), jnp.float32),
        mesh=_sc_mesh(),
        scratch_types=[
            pltpu.VMEM((GK, 16), jnp.float32),
            pltpu.VMEM((GK,), jnp.int32),
            pltpu.VMEM_SHARED((N, 16), jnp.float32),
        ],
        compiler_params=pltpu.CompilerParams(use_tc_tiling_on_sc=False),
    )


def _sc_counts_body(dst_hbm, ones_hbm, zeros_hbm, cnt_hbm, ones_v, idx_v, tbl):
    c = lax.axis_index("c")
    s = lax.axis_index("s")
    wid = s * 2 + c
    base = wid * EPT
    pltpu.sync_copy(ones_hbm, ones_v)

    @pl.when(s == 0)
    def _zero():
        pltpu.sync_copy(zeros_hbm, tbl)

    plsc.subcore_barrier()

    @pl.loop(0, EPT, step=GK)
    def _chunk(off):
        pltpu.sync_copy(dst_hbm.at[pl.ds(base + off, GK)], idx_v)
        pltpu.sync_copy(ones_v, tbl.at[idx_v], add=True)

    plsc.subcore_barrier()

    @pl.when(s < N // NPS)
    def _writeback():
        pltpu.sync_copy(tbl.at[pl.ds(s * NPS, NPS)],
                        cnt_hbm.at[c, pl.ds(s * NPS, NPS)])


def kernel(node_features, edge_index, edge_weights, time_steps, params):
    p = params
    src = edge_index[0]
    dst = edge_index[1]
    ew2d = edge_weights.reshape(E, 1)
    r = lambda a: a.reshape(1, -1)

    ones16 = jnp.ones((GK, 16), jnp.float32)
    zeros16 = jnp.zeros((N, 16), jnp.float32)
    zeros_cw = jnp.zeros((N, CW), jnp.float32)

    x = _encode(node_features, time_steps,
                p["enc_W"], r(p["enc_b"]), r(p["enc_g"]), r(p["enc_beta"]),
                r(p["te_W"][:, 0]), r(p["te_b"]), r(p["te_g"]), r(p["te_beta"]))
    cnts = _sc_counts_kernel()(dst, ones16, zeros16)

    for blk in p["blocks"]:
        ws = blk["msg_W"][:, :H]
        wd = blk["msg_W"][:, H:]
        a, b = _ab(x, ws, wd, r(blk["msg_b"]))
        g1, g2 = _sc_gather_kernel()(a, b, src, dst)
        em = _edge_math(g1, g2, ew2d, r(blk["msg_g"]), r(blk["msg_beta"]))
        msum = _sc_scatter_kernel()(em, dst, zeros_cw)
        x = _update(x, msum, cnts, blk["upd_W"], r(blk["upd_b"]),
                    r(blk["upd_g"]), r(blk["upd_beta"]),
                    r(blk["gate_W"][0]), blk["gate_b"].reshape(1, 1))

    return _out_proj(x, p["out_W"], r(p["out_b"]))


# speedup vs baseline: 1.3822x; 1.3822x over previous
import functools

import jax
import jax.numpy as jnp
from jax import lax
from jax.experimental import pallas as pl
from jax.experimental.pallas import tpu as pltpu
from jax.experimental.pallas import tpu_sc as plsc

N = 10000
E = 160000
FEAT = 256
HID = 512
TDIM = 128
H = HID + TDIM

NB = 1000
EB = 2000
PW = H // 2
PWP = 384
NCHUNK = 5
CW = H // NCHUNK

NUM_TILES = 32
EPT = E // NUM_TILES
GK = 40
EPC = E // 2
EPS = EPC // 16
SK = 40
NPS = 1000

_PREC = lax.Precision.HIGHEST


def _dotT(x, w):
    return lax.dot_general(x, w, (((1,), (1,)), ((), ())), precision=_PREC)


def _ln(v, g, b, eps=1e-5):
    m = jnp.mean(v, axis=-1, keepdims=True)
    var = jnp.mean((v - m) ** 2, axis=-1, keepdims=True)
    return (v - m) * lax.rsqrt(var + eps) * g + b


def _encode_body(nf_ref, ts_ref, ew_ref, eb_ref, eg_ref, ebeta_ref,
                 tw_ref, tb_ref, tg_ref, tbeta_ref, out_ref):
    h = _dotT(nf_ref[...], ew_ref[...]) + eb_ref[...]
    h = jax.nn.relu(_ln(h, eg_ref[...], ebeta_ref[...]))
    t = ts_ref[...] * tw_ref[...] + tb_ref[...]
    t = jax.nn.relu(_ln(t, tg_ref[...], tbeta_ref[...]))
    out_ref[:, :HID] = h
    out_ref[:, HID:] = t


def _encode(nf, ts, ew, eb, eg, ebeta, tw, tb, tg, tbeta):
    row = lambda d: pl.BlockSpec((1, d), lambda i: (0, 0))
    return pl.pallas_call(
        _encode_body,
        grid=(N // NB,),
        in_specs=[
            pl.BlockSpec((NB, FEAT), lambda i: (i, 0)),
            pl.BlockSpec((NB, 1), lambda i: (i, 0)),
            pl.BlockSpec((HID, FEAT), lambda i: (0, 0)),
            row(HID), row(HID), row(HID),
            row(TDIM), row(TDIM), row(TDIM), row(TDIM),
        ],
        out_specs=pl.BlockSpec((NB, H), lambda i: (i, 0)),
        out_shape=jax.ShapeDtypeStruct((N, H), jnp.float32),
    )(nf, ts, ew, eb, eg, ebeta, tw, tb, tg, tbeta)


def _pack2(v):
    lo = lax.bitcast_convert_type(v[:, :PW].astype(jnp.bfloat16), jnp.uint16)
    hi = lax.bitcast_convert_type(v[:, PW:].astype(jnp.bfloat16), jnp.uint16)
    packed = lo.astype(jnp.uint32) | (hi.astype(jnp.uint32) << 16)
    return lax.bitcast_convert_type(packed, jnp.float32)


def _unpack2(v):
    u = lax.bitcast_convert_type(v, jnp.uint32)
    lo = lax.bitcast_convert_type((u & 0xFFFF).astype(jnp.uint16), jnp.bfloat16)
    hi = lax.bitcast_convert_type((u >> 16).astype(jnp.uint16), jnp.bfloat16)
    return jnp.concatenate([lo.astype(jnp.float32), hi.astype(jnp.float32)],
                           axis=-1)


def _ab_body(x_ref, ws_ref, wd_ref, mb_ref, a_ref, b_ref):
    x = x_ref[...]
    a_ref[:, :PW] = _pack2(_dotT(x, ws_ref[...]))
    b_ref[:, :PW] = _pack2(_dotT(x, wd_ref[...]) + mb_ref[...])


def _ab(x, ws, wd, mb):
    return pl.pallas_call(
        _ab_body,
        grid=(N // NB,),
        in_specs=[
            pl.BlockSpec((NB, H), lambda i: (i, 0)),
            pl.BlockSpec((H, H), lambda i: (0, 0)),
            pl.BlockSpec((H, H), lambda i: (0, 0)),
            pl.BlockSpec((1, H), lambda i: (0, 0)),
        ],
        out_specs=[pl.BlockSpec((NB, PWP), lambda i: (i, 0)),
                   pl.BlockSpec((NB, PWP), lambda i: (i, 0))],
        out_shape=[jax.ShapeDtypeStruct((N, PWP), jnp.float32),
                   jax.ShapeDtypeStruct((N, PWP), jnp.float32)],
    )(x, ws, wd, mb)


def _edge_math_body(g1_ref, g2_ref, w_ref, mg_ref, mbeta_ref, out_ref):
    s = _unpack2(g1_ref[:, :PW]) + _unpack2(g2_ref[:, :PW])
    y = jax.nn.relu(_ln(s, mg_ref[...], mbeta_ref[...])) * w_ref[...]
    for c in range(NCHUNK):
        out_ref[c] = y[:, c * CW:(c + 1) * CW]


def _edge_math(g1, g2, w, mg, mbeta):
    return pl.pallas_call(
        _edge_math_body,
        grid=(E // EB,),
        in_specs=[
            pl.BlockSpec((EB, PWP), lambda i: (i, 0)),
            pl.BlockSpec((EB, PWP), lambda i: (i, 0)),
            pl.BlockSpec((EB, 1), lambda i: (i, 0)),
            pl.BlockSpec((1, H), lambda i: (0, 0)),
            pl.BlockSpec((1, H), lambda i: (0, 0)),
        ],
        out_specs=pl.BlockSpec((NCHUNK, EB, CW), lambda i: (0, i, 0)),
        out_shape=jax.ShapeDtypeStruct((NCHUNK, E, CW), jnp.float32),
    )(g1, g2, w, mg, mbeta)


def _update_body(x_ref, ms_ref, cnt_ref, uw_ref, ub_ref, ug_ref, ubeta_ref,
                 gw_ref, gb_ref, out_ref):
    x = x_ref[...]
    msum = jnp.concatenate(
        [ms_ref[0, c] + ms_ref[1, c] for c in range(NCHUNK)], axis=-1)
    cnt = cnt_ref[0, :, 0:1] + cnt_ref[1, :, 0:1]
    valid = (cnt > 0).astype(jnp.float32)
    messages = msum / (cnt + 1e-8) * valid
    tw = jax.nn.sigmoid(jnp.sum(x * gw_ref[...], axis=-1, keepdims=True)
                        + gb_ref[...])
    combined = jnp.concatenate([x, messages], axis=-1)
    h_new = _dotT(combined, uw_ref[...]) + ub_ref[...]
    h_new = jax.nn.relu(_ln(h_new, ug_ref[...], ubeta_ref[...]))
    out_ref[...] = tw * h_new + (1.0 - tw) * x


def _update(x, msum, cnts, uw, ub, ug, ubeta, gw, gb):
    return pl.pallas_call(
        _update_body,
        grid=(N // NB,),
        in_specs=[
            pl.BlockSpec((NB, H), lambda i: (i, 0)),
            pl.BlockSpec((2, NCHUNK, NB, CW), lambda i: (0, 0, i, 0)),
            pl.BlockSpec((2, NB, CW), lambda i: (0, i, 0)),
            pl.BlockSpec((H, 2 * H), lambda i: (0, 0)),
            pl.BlockSpec((1, H), lambda i: (0, 0)),
            pl.BlockSpec((1, H), lambda i: (0, 0)),
            pl.BlockSpec((1, H), lambda i: (0, 0)),
            pl.BlockSpec((1, H), lambda i: (0, 0)),
            pl.BlockSpec((1, 1), lambda i: (0, 0)),
        ],
        out_specs=pl.BlockSpec((NB, H), lambda i: (i, 0)),
        out_shape=jax.ShapeDtypeStruct((N, H), jnp.float32),
    )(x, msum, cnts, uw, ub, ug, ubeta, gw, gb)


def _out_body(x_ref, ow_ref, ob_ref, out_ref):
    o = _dotT(x_ref[...], ow_ref[...]) + ob_ref[...]
    nrm = jnp.sqrt(jnp.sum(o * o, axis=-1, keepdims=True))
    out_ref[...] = o / jnp.maximum(nrm, 1e-12)


def _out_proj(x, ow, ob):
    return pl.pallas_call(
        _out_body,
        grid=(N // NB,),
        in_specs=[
            pl.BlockSpec((NB, H), lambda i: (i, 0)),
            pl.BlockSpec((FEAT, H), lambda i: (0, 0)),
            pl.BlockSpec((1, FEAT), lambda i: (0, 0)),
        ],
        out_specs=pl.BlockSpec((NB, FEAT), lambda i: (i, 0)),
        out_shape=jax.ShapeDtypeStruct((N, FEAT), jnp.float32),
    )(x, ow, ob)


@functools.cache
def _sc_mesh():
    return plsc.VectorSubcoreMesh(core_axis_name="c", subcore_axis_name="s")


@functools.cache
def _sc_gather_kernel():
    return pl.kernel(
        _sc_gather_body,
        out_type=[jax.ShapeDtypeStruct((E, PWP), jnp.float32),
                  jax.ShapeDtypeStruct((E, PWP), jnp.float32)],
        mesh=_sc_mesh(),
        scratch_types=[
            pltpu.VMEM((EPT,), jnp.int32),
            pltpu.VMEM((EPT,), jnp.int32),
            pltpu.VMEM((GK, PWP), jnp.float32),
            pltpu.VMEM((GK, PWP), jnp.float32),
            pltpu.SemaphoreType.DMA,
        ],
    )


def _sc_gather_body(a_hbm, b_hbm, src_hbm, dst_hbm, g1_hbm, g2_hbm,
                    idx_s, idx_d, rows_a, rows_b, sem):
    c = lax.axis_index("c")
    s = lax.axis_index("s")
    wid = s * 2 + c
    base = wid * EPT
    pltpu.sync_copy(src_hbm.at[pl.ds(base, EPT)], idx_s)
    pltpu.sync_copy(dst_hbm.at[pl.ds(base, EPT)], idx_d)

    @pl.loop(0, EPT, step=GK)
    def _chunk(off):
        ca = pltpu.async_copy(a_hbm.at[idx_s.at[pl.ds(off, GK)]], rows_a, sem)
        cb = pltpu.async_copy(b_hbm.at[idx_d.at[pl.ds(off, GK)]], rows_b, sem)
        ca.wait()
        cb.wait()
        pltpu.sync_copy(rows_a, g1_hbm.at[pl.ds(base + off, GK)])
        pltpu.sync_copy(rows_b, g2_hbm.at[pl.ds(base + off, GK)])


@functools.cache
def _sc_scatter_kernel():
    return pl.kernel(
        _sc_scatter_body,
        out_type=jax.ShapeDtypeStruct((2, NCHUNK, N, CW), jnp.float32),
        mesh=_sc_mesh(),
        scratch_types=[
            pltpu.VMEM((SK, CW), jnp.float32),
            pltpu.VMEM((SK,), jnp.int32),
            pltpu.VMEM_SHARED((N, CW), jnp.float32),
        ],
    )


def _sc_scatter_body(em_hbm, dst_hbm, zeros_hbm, msum_hbm, rows_v, idx_v, tbl):
    c = lax.axis_index("c")
    s = lax.axis_index("s")
    base = c * EPC + s * EPS
    for chunk in range(NCHUNK):
        @pl.when(s == 0)
        def _zero():
            pltpu.sync_copy(zeros_hbm, tbl)

        plsc.subcore_barrier()

        @pl.loop(0, EPS, step=SK)
        def _chunk_loop(off):
            pltpu.sync_copy(dst_hbm.at[pl.ds(base + off, SK)], idx_v)
            pltpu.sync_copy(em_hbm.at[chunk, pl.ds(base + off, SK)], rows_v)
            pltpu.sync_copy(rows_v, tbl.at[idx_v], add=True)

        plsc.subcore_barrier()

        @pl.when(s < N // NPS)
        def _writeback():
            pltpu.sync_copy(tbl.at[pl.ds(s * NPS, NPS)],
                            msum_hbm.at[c, chunk, pl.ds(s * NPS, NPS)])

        plsc.subcore_barrier()


@functools.cache
def _sc_counts_kernel():
    return pl.kernel(
        _sc_counts_body,
        out_type=jax.ShapeDtypeStruct((2, N, CW), jnp.float32),
        mesh=_sc_mesh(),
        scratch_types=[
            pltpu.VMEM((GK, CW), jnp.float32),
            pltpu.VMEM((GK,), jnp.int32),
            pltpu.VMEM_SHARED((N, CW), jnp.float32),
        ],
    )


def _sc_counts_body(dst_hbm, ones_hbm, zeros_hbm, cnt_hbm, ones_v, idx_v, tbl):
    c = lax.axis_index("c")
    s = lax.axis_index("s")
    wid = s * 2 + c
    base = wid * EPT
    pltpu.sync_copy(ones_hbm, ones_v)

    @pl.when(s == 0)
    def _zero():
        pltpu.sync_copy(zeros_hbm, tbl)

    plsc.subcore_barrier()

    @pl.loop(0, EPT, step=GK)
    def _chunk(off):
        pltpu.sync_copy(dst_hbm.at[pl.ds(base + off, GK)], idx_v)
        pltpu.sync_copy(ones_v, tbl.at[idx_v], add=True)

    plsc.subcore_barrier()

    @pl.when(s < N // NPS)
    def _writeback():
        pltpu.sync_copy(tbl.at[pl.ds(s * NPS, NPS)],
                        cnt_hbm.at[c, pl.ds(s * NPS, NPS)])


def kernel(node_features, edge_index, edge_weights, time_steps, params):
    p = params
    src = edge_index[0]
    dst = edge_index[1]
    ew2d = edge_weights.reshape(E, 1)
    r = lambda a: a.reshape(1, -1)

    ones_cw = jnp.ones((GK, CW), jnp.float32)
    zeros_cw = jnp.zeros((N, CW), jnp.float32)

    x = _encode(node_features, time_steps,
                p["enc_W"], r(p["enc_b"]), r(p["enc_g"]), r(p["enc_beta"]),
                r(p["te_W"][:, 0]), r(p["te_b"]), r(p["te_g"]), r(p["te_beta"]))
    cnts = _sc_counts_kernel()(dst, ones_cw, zeros_cw)

    for blk in p["blocks"]:
        ws = blk["msg_W"][:, :H]
        wd = blk["msg_W"][:, H:]
        a, b = _ab(x, ws, wd, r(blk["msg_b"]))
        g1, g2 = _sc_gather_kernel()(a, b, src, dst)
        em = _edge_math(g1, g2, ew2d, r(blk["msg_g"]), r(blk["msg_beta"]))
        msum = _sc_scatter_kernel()(em, dst, zeros_cw)
        x = _update(x, msum, cnts, blk["upd_W"], r(blk["upd_b"]),
                    r(blk["upd_g"]), r(blk["upd_beta"]),
                    r(blk["gate_W"][0]), blk["gate_b"].reshape(1, 1))

    return _out_proj(x, p["out_W"], r(p["out_b"]))


# speedup vs baseline: 1.9324x; 1.3980x over previous
import functools

import jax
import jax.numpy as jnp
from jax import lax
from jax.experimental import pallas as pl
from jax.experimental.pallas import tpu as pltpu
from jax.experimental.pallas import tpu_sc as plsc

N = 10000
E = 160000
FEAT = 256
HID = 512
TDIM = 128
H = HID + TDIM

NB = 1000
EB = 2000
PW = H // 2
PWP = 384
NCHUNK = 5
CW = H // NCHUNK

NUM_TILES = 32
EPT = E // NUM_TILES
GK = 40
EPC = E // 2
EPS = EPC // 16
SK = 40
NPS = 1000

_PREC = lax.Precision.HIGHEST


def _dotT(x, w):
    return lax.dot_general(x, w, (((1,), (1,)), ((), ())), precision=_PREC)


def _ln(v, g, b, eps=1e-5):
    m = jnp.mean(v, axis=-1, keepdims=True)
    var = jnp.mean((v - m) ** 2, axis=-1, keepdims=True)
    return (v - m) * lax.rsqrt(var + eps) * g + b


def _encode_body(nf_ref, ts_ref, ew_ref, eb_ref, eg_ref, ebeta_ref,
                 tw_ref, tb_ref, tg_ref, tbeta_ref, out_ref):
    h = _dotT(nf_ref[...], ew_ref[...]) + eb_ref[...]
    h = jax.nn.relu(_ln(h, eg_ref[...], ebeta_ref[...]))
    t = ts_ref[...] * tw_ref[...] + tb_ref[...]
    t = jax.nn.relu(_ln(t, tg_ref[...], tbeta_ref[...]))
    out_ref[:, :HID] = h
    out_ref[:, HID:] = t


def _encode(nf, ts, ew, eb, eg, ebeta, tw, tb, tg, tbeta):
    row = lambda d: pl.BlockSpec((1, d), lambda i: (0, 0))
    return pl.pallas_call(
        _encode_body,
        grid=(N // NB,),
        in_specs=[
            pl.BlockSpec((NB, FEAT), lambda i: (i, 0)),
            pl.BlockSpec((NB, 1), lambda i: (i, 0)),
            pl.BlockSpec((HID, FEAT), lambda i: (0, 0)),
            row(HID), row(HID), row(HID),
            row(TDIM), row(TDIM), row(TDIM), row(TDIM),
        ],
        out_specs=pl.BlockSpec((NB, H), lambda i: (i, 0)),
        out_shape=jax.ShapeDtypeStruct((N, H), jnp.float32),
    )(nf, ts, ew, eb, eg, ebeta, tw, tb, tg, tbeta)


def _pack2(v):
    lo = lax.bitcast_convert_type(v[:, :PW].astype(jnp.bfloat16), jnp.uint16)
    hi = lax.bitcast_convert_type(v[:, PW:].astype(jnp.bfloat16), jnp.uint16)
    packed = lo.astype(jnp.uint32) | (hi.astype(jnp.uint32) << 16)
    return lax.bitcast_convert_type(packed, jnp.float32)


def _unpack2(v):
    u = lax.bitcast_convert_type(v, jnp.uint32)
    lo = lax.bitcast_convert_type((u & 0xFFFF).astype(jnp.uint16), jnp.bfloat16)
    hi = lax.bitcast_convert_type((u >> 16).astype(jnp.uint16), jnp.bfloat16)
    return jnp.concatenate([lo.astype(jnp.float32), hi.astype(jnp.float32)],
                           axis=-1)


def _dotT_fast(x, w):
    return lax.dot_general(x, w, (((1,), (1,)), ((), ())),
                           precision=lax.Precision.DEFAULT)


def _ab_body(x_ref, ws_ref, wd_ref, mb_ref, a_ref, b_ref):
    x = x_ref[...]
    a_ref[:, :PW] = _pack2(_dotT_fast(x, ws_ref[...]))
    b_ref[:, :PW] = _pack2(_dotT_fast(x, wd_ref[...]) + mb_ref[...])


def _ab(x, ws, wd, mb):
    return pl.pallas_call(
        _ab_body,
        grid=(N // NB,),
        in_specs=[
            pl.BlockSpec((NB, H), lambda i: (i, 0)),
            pl.BlockSpec((H, H), lambda i: (0, 0)),
            pl.BlockSpec((H, H), lambda i: (0, 0)),
            pl.BlockSpec((1, H), lambda i: (0, 0)),
        ],
        out_specs=[pl.BlockSpec((NB, PWP), lambda i: (i, 0)),
                   pl.BlockSpec((NB, PWP), lambda i: (i, 0))],
        out_shape=[jax.ShapeDtypeStruct((N, PWP), jnp.float32),
                   jax.ShapeDtypeStruct((N, PWP), jnp.float32)],
    )(x, ws, wd, mb)


def _edge_math_body(g1_ref, g2_ref, w_ref, mg_ref, mbeta_ref, out_ref):
    s = _unpack2(g1_ref[:, :PW]) + _unpack2(g2_ref[:, :PW])
    w_col = jnp.swapaxes(w_ref[0], 0, 1)
    y = jax.nn.relu(_ln(s, mg_ref[...], mbeta_ref[...])) * w_col
    for c in range(NCHUNK):
        out_ref[c] = y[:, c * CW:(c + 1) * CW]


def _edge_math(g1, g2, w, mg, mbeta):
    return pl.pallas_call(
        _edge_math_body,
        grid=(E // EB,),
        in_specs=[
            pl.BlockSpec((EB, PWP), lambda i: (i, 0)),
            pl.BlockSpec((EB, PWP), lambda i: (i, 0)),
            pl.BlockSpec((1, 1, EB), lambda i: (i, 0, 0)),
            pl.BlockSpec((1, H), lambda i: (0, 0)),
            pl.BlockSpec((1, H), lambda i: (0, 0)),
        ],
        out_specs=pl.BlockSpec((NCHUNK, EB, CW), lambda i: (0, i, 0)),
        out_shape=jax.ShapeDtypeStruct((NCHUNK, E, CW), jnp.float32),
    )(g1, g2, w, mg, mbeta)


def _update_body(x_ref, ms_ref, cnt_ref, uw_ref, ub_ref, ug_ref, ubeta_ref,
                 gw_ref, gb_ref, out_ref):
    x = x_ref[...]
    msum = jnp.concatenate(
        [ms_ref[0, c] + ms_ref[1, c] for c in range(NCHUNK)], axis=-1)
    cnt = cnt_ref[0, :, 0:1] + cnt_ref[1, :, 0:1]
    valid = (cnt > 0).astype(jnp.float32)
    messages = msum / (cnt + 1e-8) * valid
    tw = jax.nn.sigmoid(jnp.sum(x * gw_ref[...], axis=-1, keepdims=True)
                        + gb_ref[...])
    combined = jnp.concatenate([x, messages], axis=-1)
    h_new = _dotT(combined, uw_ref[...]) + ub_ref[...]
    h_new = jax.nn.relu(_ln(h_new, ug_ref[...], ubeta_ref[...]))
    out_ref[...] = tw * h_new + (1.0 - tw) * x


def _update(x, msum, cnts, uw, ub, ug, ubeta, gw, gb):
    return pl.pallas_call(
        _update_body,
        grid=(N // NB,),
        in_specs=[
            pl.BlockSpec((NB, H), lambda i: (i, 0)),
            pl.BlockSpec((2, NCHUNK, NB, CW), lambda i: (0, 0, i, 0)),
            pl.BlockSpec((2, NB, CW), lambda i: (0, i, 0)),
            pl.BlockSpec((H, 2 * H), lambda i: (0, 0)),
            pl.BlockSpec((1, H), lambda i: (0, 0)),
            pl.BlockSpec((1, H), lambda i: (0, 0)),
            pl.BlockSpec((1, H), lambda i: (0, 0)),
            pl.BlockSpec((1, H), lambda i: (0, 0)),
            pl.BlockSpec((1, 1), lambda i: (0, 0)),
        ],
        out_specs=pl.BlockSpec((NB, H), lambda i: (i, 0)),
        out_shape=jax.ShapeDtypeStruct((N, H), jnp.float32),
    )(x, msum, cnts, uw, ub, ug, ubeta, gw, gb)


def _out_body(x_ref, ow_ref, ob_ref, out_ref):
    o = _dotT(x_ref[...], ow_ref[...]) + ob_ref[...]
    nrm = jnp.sqrt(jnp.sum(o * o, axis=-1, keepdims=True))
    out_ref[...] = o / jnp.maximum(nrm, 1e-12)


def _out_proj(x, ow, ob):
    return pl.pallas_call(
        _out_body,
        grid=(N // NB,),
        in_specs=[
            pl.BlockSpec((NB, H), lambda i: (i, 0)),
            pl.BlockSpec((FEAT, H), lambda i: (0, 0)),
            pl.BlockSpec((1, FEAT), lambda i: (0, 0)),
        ],
        out_specs=pl.BlockSpec((NB, FEAT), lambda i: (i, 0)),
        out_shape=jax.ShapeDtypeStruct((N, FEAT), jnp.float32),
    )(x, ow, ob)


@functools.cache
def _sc_mesh():
    return plsc.VectorSubcoreMesh(core_axis_name="c", subcore_axis_name="s")


@functools.cache
def _sc_gather_kernel():
    return pl.kernel(
        _sc_gather_body,
        out_type=[jax.ShapeDtypeStruct((E, PWP), jnp.float32),
                  jax.ShapeDtypeStruct((E, PWP), jnp.float32)],
        mesh=_sc_mesh(),
        scratch_types=[
            pltpu.VMEM((EPT,), jnp.int32),
            pltpu.VMEM((EPT,), jnp.int32),
            pltpu.VMEM((GK, PWP), jnp.float32),
            pltpu.VMEM((GK, PWP), jnp.float32),
            pltpu.SemaphoreType.DMA,
        ],
    )


def _sc_gather_body(a_hbm, b_hbm, src_hbm, dst_hbm, g1_hbm, g2_hbm,
                    idx_s, idx_d, rows_a, rows_b, sem):
    c = lax.axis_index("c")
    s = lax.axis_index("s")
    wid = s * 2 + c
    base = wid * EPT
    pltpu.sync_copy(src_hbm.at[pl.ds(base, EPT)], idx_s)
    pltpu.sync_copy(dst_hbm.at[pl.ds(base, EPT)], idx_d)

    @pl.loop(0, EPT, step=GK)
    def _chunk(off):
        ca = pltpu.async_copy(a_hbm.at[idx_s.at[pl.ds(off, GK)]], rows_a, sem)
        cb = pltpu.async_copy(b_hbm.at[idx_d.at[pl.ds(off, GK)]], rows_b, sem)
        ca.wait()
        cb.wait()
        pltpu.sync_copy(rows_a, g1_hbm.at[pl.ds(base + off, GK)])
        pltpu.sync_copy(rows_b, g2_hbm.at[pl.ds(base + off, GK)])


IPS = EPS // SK


@functools.cache
def _sc_scatter_kernel():
    return pl.kernel(
        _sc_scatter_body,
        out_type=jax.ShapeDtypeStruct((2, NCHUNK, N, CW), jnp.float32),
        mesh=_sc_mesh(),
        scratch_types=[
            pltpu.VMEM((IPS, SK), jnp.int32),
            pltpu.VMEM((SK, CW), jnp.float32),
            pltpu.VMEM((SK, CW), jnp.float32),
            pltpu.SemaphoreType.DMA,
            pltpu.SemaphoreType.DMA,
            pltpu.VMEM_SHARED((N, CW), jnp.float32),
        ],
    )


def _sc_scatter_body(em_hbm, dst2d_hbm, zeros_hbm, msum_hbm,
                     idx2d, rows0, rows1, sem0, sem1, tbl):
    c = lax.axis_index("c")
    s = lax.axis_index("s")
    base = c * EPC + s * EPS
    pltpu.sync_copy(dst2d_hbm.at[c * 16 + s], idx2d)
    for chunk in range(NCHUNK):
        @pl.when(s == 0)
        def _zero():
            pltpu.sync_copy(zeros_hbm, tbl)

        plsc.subcore_barrier()

        @pl.loop(0, IPS - 1, step=2)
        def _chunk_loop(j):
            c0 = pltpu.async_copy(
                em_hbm.at[chunk, pl.ds(base + j * SK, SK)], rows0, sem0)
            c1 = pltpu.async_copy(
                em_hbm.at[chunk, pl.ds(base + (j + 1) * SK, SK)], rows1, sem1)
            c0.wait()
            pltpu.sync_copy(rows0, tbl.at[idx2d.at[j]], add=True)
            c1.wait()
            pltpu.sync_copy(rows1, tbl.at[idx2d.at[j + 1]], add=True)

        pltpu.async_copy(
            em_hbm.at[chunk, pl.ds(base + (IPS - 1) * SK, SK)], rows0,
            sem0).wait()
        pltpu.sync_copy(rows0, tbl.at[idx2d.at[IPS - 1]], add=True)

        plsc.subcore_barrier()

        @pl.when(s < N // NPS)
        def _writeback():
            pltpu.sync_copy(tbl.at[pl.ds(s * NPS, NPS)],
                            msum_hbm.at[c, chunk, pl.ds(s * NPS, NPS)])

        plsc.subcore_barrier()


@functools.cache
def _sc_counts_kernel():
    return pl.kernel(
        _sc_counts_body,
        out_type=jax.ShapeDtypeStruct((2, N, CW), jnp.float32),
        mesh=_sc_mesh(),
        scratch_types=[
            pltpu.VMEM((GK, CW), jnp.float32),
            pltpu.VMEM((GK,), jnp.int32),
            pltpu.VMEM_SHARED((N, CW), jnp.float32),
        ],
    )


def _sc_counts_body(dst_hbm, ones_hbm, zeros_hbm, cnt_hbm, ones_v, idx_v, tbl):
    c = lax.axis_index("c")
    s = lax.axis_index("s")
    wid = s * 2 + c
    base = wid * EPT
    pltpu.sync_copy(ones_hbm, ones_v)

    @pl.when(s == 0)
    def _zero():
        pltpu.sync_copy(zeros_hbm, tbl)

    plsc.subcore_barrier()

    @pl.loop(0, EPT, step=GK)
    def _chunk(off):
        pltpu.sync_copy(dst_hbm.at[pl.ds(base + off, GK)], idx_v)
        pltpu.sync_copy(ones_v, tbl.at[idx_v], add=True)

    plsc.subcore_barrier()

    @pl.when(s < N // NPS)
    def _writeback():
        pltpu.sync_copy(tbl.at[pl.ds(s * NPS, NPS)],
                        cnt_hbm.at[c, pl.ds(s * NPS, NPS)])


def kernel(node_features, edge_index, edge_weights, time_steps, params):
    p = params
    src = edge_index[0]
    dst = edge_index[1]
    ew3d = edge_weights.reshape(E // EB, 1, EB)
    dst3d = dst.reshape(NUM_TILES, IPS, SK)
    r = lambda a: a.reshape(1, -1)

    ones_cw = jnp.ones((GK, CW), jnp.float32)
    zeros_cw = jnp.zeros((N, CW), jnp.float32)

    x = _encode(node_features, time_steps,
                p["enc_W"], r(p["enc_b"]), r(p["enc_g"]), r(p["enc_beta"]),
                r(p["te_W"][:, 0]), r(p["te_b"]), r(p["te_g"]), r(p["te_beta"]))
    cnts = _sc_counts_kernel()(dst, ones_cw, zeros_cw)

    for blk in p["blocks"]:
        ws = blk["msg_W"][:, :H]
        wd = blk["msg_W"][:, H:]
        a, b = _ab(x, ws, wd, r(blk["msg_b"]))
        g1, g2 = _sc_gather_kernel()(a, b, src, dst)
        em = _edge_math(g1, g2, ew3d, r(blk["msg_g"]), r(blk["msg_beta"]))
        msum = _sc_scatter_kernel()(em, dst3d, zeros_cw)
        x = _update(x, msum, cnts, blk["upd_W"], r(blk["upd_b"]),
                    r(blk["upd_g"]), r(blk["upd_beta"]),
                    r(blk["gate_W"][0]), blk["gate_b"].reshape(1, 1))

    return _out_proj(x, p["out_W"], r(p["out_b"]))


# speedup vs baseline: 2.1839x; 1.1302x over previous
import functools

import jax
import jax.numpy as jnp
from jax import lax
from jax.experimental import pallas as pl
from jax.experimental.pallas import tpu as pltpu
from jax.experimental.pallas import tpu_sc as plsc

N = 10000
E = 160000
FEAT = 256
HID = 512
TDIM = 128
H = HID + TDIM

NB = 1000
EB = 2000
PW = H // 2
PWP = 384
NCHUNK = 5
CW = H // NCHUNK

NUM_TILES = 32
EPT = E // NUM_TILES
GK = 120
GKT = 80
CK = 40
EPC = E // 2
EPS = EPC // 16
SK = 40
NPS = 1000

_PREC = lax.Precision.HIGHEST


def _dotT(x, w):
    return lax.dot_general(x, w, (((1,), (1,)), ((), ())), precision=_PREC)


def _ln(v, g, b, eps=1e-5):
    m = jnp.mean(v, axis=-1, keepdims=True)
    var = jnp.mean((v - m) ** 2, axis=-1, keepdims=True)
    return (v - m) * lax.rsqrt(var + eps) * g + b


def _encode_body(nf_ref, ts_ref, ew_ref, eb_ref, eg_ref, ebeta_ref,
                 tw_ref, tb_ref, tg_ref, tbeta_ref, out_ref):
    h = _dotT(nf_ref[...], ew_ref[...]) + eb_ref[...]
    h = jax.nn.relu(_ln(h, eg_ref[...], ebeta_ref[...]))
    t = ts_ref[...] * tw_ref[...] + tb_ref[...]
    t = jax.nn.relu(_ln(t, tg_ref[...], tbeta_ref[...]))
    out_ref[:, :HID] = h
    out_ref[:, HID:] = t


def _encode(nf, ts, ew, eb, eg, ebeta, tw, tb, tg, tbeta):
    row = lambda d: pl.BlockSpec((1, d), lambda i: (0, 0))
    return pl.pallas_call(
        _encode_body,
        grid=(N // NB,),
        in_specs=[
            pl.BlockSpec((NB, FEAT), lambda i: (i, 0)),
            pl.BlockSpec((NB, 1), lambda i: (i, 0)),
            pl.BlockSpec((HID, FEAT), lambda i: (0, 0)),
            row(HID), row(HID), row(HID),
            row(TDIM), row(TDIM), row(TDIM), row(TDIM),
        ],
        out_specs=pl.BlockSpec((NB, H), lambda i: (i, 0)),
        out_shape=jax.ShapeDtypeStruct((N, H), jnp.float32),
    )(nf, ts, ew, eb, eg, ebeta, tw, tb, tg, tbeta)


def _pack2(v):
    lo = lax.bitcast_convert_type(v[:, :PW].astype(jnp.bfloat16), jnp.uint16)
    hi = lax.bitcast_convert_type(v[:, PW:].astype(jnp.bfloat16), jnp.uint16)
    packed = lo.astype(jnp.uint32) | (hi.astype(jnp.uint32) << 16)
    return lax.bitcast_convert_type(packed, jnp.float32)


def _unpack2(v):
    u = lax.bitcast_convert_type(v, jnp.uint32)
    lo = lax.bitcast_convert_type((u & 0xFFFF).astype(jnp.uint16), jnp.bfloat16)
    hi = lax.bitcast_convert_type((u >> 16).astype(jnp.uint16), jnp.bfloat16)
    return jnp.concatenate([lo.astype(jnp.float32), hi.astype(jnp.float32)],
                           axis=-1)


def _dotT_fast(x, w):
    return lax.dot_general(x, w, (((1,), (1,)), ((), ())),
                           precision=lax.Precision.DEFAULT)


def _ab_body(x_ref, ws_ref, wd_ref, mb_ref, a_ref, b_ref):
    x = x_ref[...]
    a_ref[:, :PW] = _pack2(_dotT_fast(x, ws_ref[...]))
    b_ref[:, :PW] = _pack2(_dotT_fast(x, wd_ref[...]) + mb_ref[...])


def _ab(x, ws, wd, mb):
    return pl.pallas_call(
        _ab_body,
        grid=(N // NB,),
        in_specs=[
            pl.BlockSpec((NB, H), lambda i: (i, 0)),
            pl.BlockSpec((H, H), lambda i: (0, 0)),
            pl.BlockSpec((H, H), lambda i: (0, 0)),
            pl.BlockSpec((1, H), lambda i: (0, 0)),
        ],
        out_specs=[pl.BlockSpec((NB, PWP), lambda i: (i, 0)),
                   pl.BlockSpec((NB, PWP), lambda i: (i, 0))],
        out_shape=[jax.ShapeDtypeStruct((N, PWP), jnp.float32),
                   jax.ShapeDtypeStruct((N, PWP), jnp.float32)],
    )(x, ws, wd, mb)


def _edge_math_body(g1_ref, g2_ref, w_ref, mg_ref, mbeta_ref, out_ref):
    s = _unpack2(g1_ref[:, :PW]) + _unpack2(g2_ref[:, :PW])
    w_col = jnp.swapaxes(w_ref[0], 0, 1)
    y = jax.nn.relu(_ln(s, mg_ref[...], mbeta_ref[...])) * w_col
    for c in range(NCHUNK):
        out_ref[c] = y[:, c * CW:(c + 1) * CW]


def _edge_math(g1, g2, w, mg, mbeta):
    return pl.pallas_call(
        _edge_math_body,
        grid=(E // EB,),
        in_specs=[
            pl.BlockSpec((EB, PWP), lambda i: (i, 0)),
            pl.BlockSpec((EB, PWP), lambda i: (i, 0)),
            pl.BlockSpec((1, 1, EB), lambda i: (i, 0, 0)),
            pl.BlockSpec((1, H), lambda i: (0, 0)),
            pl.BlockSpec((1, H), lambda i: (0, 0)),
        ],
        out_specs=pl.BlockSpec((NCHUNK, EB, CW), lambda i: (0, i, 0)),
        out_shape=jax.ShapeDtypeStruct((NCHUNK, E, CW), jnp.float32),
    )(g1, g2, w, mg, mbeta)


def _update_body(x_ref, ms_ref, cnt_ref, uw_ref, ub_ref, ug_ref, ubeta_ref,
                 gw_ref, gb_ref, out_ref):
    x = x_ref[...]
    msum = jnp.concatenate(
        [ms_ref[0, c] + ms_ref[1, c] for c in range(NCHUNK)], axis=-1)
    cnt = cnt_ref[0, :, 0:1] + cnt_ref[1, :, 0:1]
    valid = (cnt > 0).astype(jnp.float32)
    messages = msum / (cnt + 1e-8) * valid
    tw = jax.nn.sigmoid(jnp.sum(x * gw_ref[...], axis=-1, keepdims=True)
                        + gb_ref[...])
    combined = jnp.concatenate([x, messages], axis=-1)
    h_new = _dotT_fast(combined, uw_ref[...]) + ub_ref[...]
    h_new = jax.nn.relu(_ln(h_new, ug_ref[...], ubeta_ref[...]))
    out_ref[...] = tw * h_new + (1.0 - tw) * x


def _update(x, msum, cnts, uw, ub, ug, ubeta, gw, gb):
    return pl.pallas_call(
        _update_body,
        grid=(N // NB,),
        in_specs=[
            pl.BlockSpec((NB, H), lambda i: (i, 0)),
            pl.BlockSpec((2, NCHUNK, NB, CW), lambda i: (0, 0, i, 0)),
            pl.BlockSpec((2, NB, CW), lambda i: (0, i, 0)),
            pl.BlockSpec((H, 2 * H), lambda i: (0, 0)),
            pl.BlockSpec((1, H), lambda i: (0, 0)),
            pl.BlockSpec((1, H), lambda i: (0, 0)),
            pl.BlockSpec((1, H), lambda i: (0, 0)),
            pl.BlockSpec((1, H), lambda i: (0, 0)),
            pl.BlockSpec((1, 1), lambda i: (0, 0)),
        ],
        out_specs=pl.BlockSpec((NB, H), lambda i: (i, 0)),
        out_shape=jax.ShapeDtypeStruct((N, H), jnp.float32),
    )(x, msum, cnts, uw, ub, ug, ubeta, gw, gb)


def _out_body(x_ref, ow_ref, ob_ref, out_ref):
    o = _dotT(x_ref[...], ow_ref[...]) + ob_ref[...]
    nrm = jnp.sqrt(jnp.sum(o * o, axis=-1, keepdims=True))
    out_ref[...] = o / jnp.maximum(nrm, 1e-12)


def _out_proj(x, ow, ob):
    return pl.pallas_call(
        _out_body,
        grid=(N // NB,),
        in_specs=[
            pl.BlockSpec((NB, H), lambda i: (i, 0)),
            pl.BlockSpec((FEAT, H), lambda i: (0, 0)),
            pl.BlockSpec((1, FEAT), lambda i: (0, 0)),
        ],
        out_specs=pl.BlockSpec((NB, FEAT), lambda i: (i, 0)),
        out_shape=jax.ShapeDtypeStruct((N, FEAT), jnp.float32),
    )(x, ow, ob)


@functools.cache
def _sc_mesh():
    return plsc.VectorSubcoreMesh(core_axis_name="c", subcore_axis_name="s")


@functools.cache
def _sc_gather_kernel():
    return pl.kernel(
        _sc_gather_body,
        out_type=[jax.ShapeDtypeStruct((E, PWP), jnp.float32),
                  jax.ShapeDtypeStruct((E, PWP), jnp.float32)],
        mesh=_sc_mesh(),
        scratch_types=[
            pltpu.VMEM((EPT,), jnp.int32),
            pltpu.VMEM((EPT,), jnp.int32),
            pltpu.VMEM((GK, PWP), jnp.float32),
            pltpu.VMEM((GK, PWP), jnp.float32),
            pltpu.SemaphoreType.DMA,
        ],
    )


def _sc_gather_body(a_hbm, b_hbm, src_hbm, dst_hbm, g1_hbm, g2_hbm,
                    idx_s, idx_d, rows_a, rows_b, sem):
    c = lax.axis_index("c")
    s = lax.axis_index("s")
    wid = s * 2 + c
    base = wid * EPT
    pltpu.sync_copy(src_hbm.at[pl.ds(base, EPT)], idx_s)
    pltpu.sync_copy(dst_hbm.at[pl.ds(base, EPT)], idx_d)

    @pl.loop(0, EPT - GKT, step=GK)
    def _chunk(off):
        ca = pltpu.async_copy(a_hbm.at[idx_s.at[pl.ds(off, GK)]], rows_a, sem)
        cb = pltpu.async_copy(b_hbm.at[idx_d.at[pl.ds(off, GK)]], rows_b, sem)
        ca.wait()
        cb.wait()
        pltpu.sync_copy(rows_a, g1_hbm.at[pl.ds(base + off, GK)])
        pltpu.sync_copy(rows_b, g2_hbm.at[pl.ds(base + off, GK)])

    toff = EPT - GKT
    ca = pltpu.async_copy(a_hbm.at[idx_s.at[pl.ds(toff, GKT)]],
                          rows_a.at[pl.ds(0, GKT)], sem)
    cb = pltpu.async_copy(b_hbm.at[idx_d.at[pl.ds(toff, GKT)]],
                          rows_b.at[pl.ds(0, GKT)], sem)
    ca.wait()
    cb.wait()
    pltpu.sync_copy(rows_a.at[pl.ds(0, GKT)], g1_hbm.at[pl.ds(base + toff, GKT)])
    pltpu.sync_copy(rows_b.at[pl.ds(0, GKT)], g2_hbm.at[pl.ds(base + toff, GKT)])


IPS = EPS // SK


@functools.cache
def _sc_scatter_kernel():
    return pl.kernel(
        _sc_scatter_body,
        out_type=jax.ShapeDtypeStruct((2, NCHUNK, N, CW), jnp.float32),
        mesh=_sc_mesh(),
        scratch_types=[
            pltpu.VMEM((IPS, SK), jnp.int32),
            pltpu.VMEM((SK, CW), jnp.float32),
            pltpu.VMEM((SK, CW), jnp.float32),
            pltpu.SemaphoreType.DMA,
            pltpu.SemaphoreType.DMA,
            pltpu.VMEM_SHARED((N, CW), jnp.float32),
        ],
    )


def _sc_scatter_body(em_hbm, dst2d_hbm, zeros_hbm, msum_hbm,
                     idx2d, rows0, rows1, sem0, sem1, tbl):
    c = lax.axis_index("c")
    s = lax.axis_index("s")
    base = c * EPC + s * EPS
    pltpu.sync_copy(dst2d_hbm.at[c * 16 + s], idx2d)
    for chunk in range(NCHUNK):
        @pl.when(s == 0)
        def _zero():
            pltpu.sync_copy(zeros_hbm, tbl)

        plsc.subcore_barrier()

        @pl.loop(0, IPS - 1, step=2)
        def _chunk_loop(j):
            c0 = pltpu.async_copy(
                em_hbm.at[chunk, pl.ds(base + j * SK, SK)], rows0, sem0)
            c1 = pltpu.async_copy(
                em_hbm.at[chunk, pl.ds(base + (j + 1) * SK, SK)], rows1, sem1)
            c0.wait()
            pltpu.sync_copy(rows0, tbl.at[idx2d.at[j]], add=True)
            c1.wait()
            pltpu.sync_copy(rows1, tbl.at[idx2d.at[j + 1]], add=True)

        pltpu.async_copy(
            em_hbm.at[chunk, pl.ds(base + (IPS - 1) * SK, SK)], rows0,
            sem0).wait()
        pltpu.sync_copy(rows0, tbl.at[idx2d.at[IPS - 1]], add=True)

        plsc.subcore_barrier()

        @pl.when(s < N // NPS)
        def _writeback():
            pltpu.sync_copy(tbl.at[pl.ds(s * NPS, NPS)],
                            msum_hbm.at[c, chunk, pl.ds(s * NPS, NPS)])

        plsc.subcore_barrier()


@functools.cache
def _sc_counts_kernel():
    return pl.kernel(
        _sc_counts_body,
        out_type=jax.ShapeDtypeStruct((2, N, CW), jnp.float32),
        mesh=_sc_mesh(),
        scratch_types=[
            pltpu.VMEM((CK, CW), jnp.float32),
            pltpu.VMEM((CK,), jnp.int32),
            pltpu.VMEM_SHARED((N, CW), jnp.float32),
        ],
    )


def _sc_counts_body(dst_hbm, ones_hbm, zeros_hbm, cnt_hbm, ones_v, idx_v, tbl):
    c = lax.axis_index("c")
    s = lax.axis_index("s")
    wid = s * 2 + c
    base = wid * EPT
    pltpu.sync_copy(ones_hbm, ones_v)

    @pl.when(s == 0)
    def _zero():
        pltpu.sync_copy(zeros_hbm, tbl)

    plsc.subcore_barrier()

    @pl.loop(0, EPT, step=CK)
    def _chunk(off):
        pltpu.sync_copy(dst_hbm.at[pl.ds(base + off, CK)], idx_v)
        pltpu.sync_copy(ones_v, tbl.at[idx_v], add=True)

    plsc.subcore_barrier()

    @pl.when(s < N // NPS)
    def _writeback():
        pltpu.sync_copy(tbl.at[pl.ds(s * NPS, NPS)],
                        cnt_hbm.at[c, pl.ds(s * NPS, NPS)])


def kernel(node_features, edge_index, edge_weights, time_steps, params):
    p = params
    src = edge_index[0]
    dst = edge_index[1]
    ew3d = edge_weights.reshape(E // EB, 1, EB)
    dst3d = dst.reshape(NUM_TILES, IPS, SK)
    r = lambda a: a.reshape(1, -1)

    ones_cw = jnp.ones((CK, CW), jnp.float32)
    zeros_cw = jnp.zeros((N, CW), jnp.float32)

    x = _encode(node_features, time_steps,
                p["enc_W"], r(p["enc_b"]), r(p["enc_g"]), r(p["enc_beta"]),
                r(p["te_W"][:, 0]), r(p["te_b"]), r(p["te_g"]), r(p["te_beta"]))
    cnts = _sc_counts_kernel()(dst, ones_cw, zeros_cw)

    for blk in p["blocks"]:
        ws = blk["msg_W"][:, :H]
        wd = blk["msg_W"][:, H:]
        a, b = _ab(x, ws, wd, r(blk["msg_b"]))
        g1, g2 = _sc_gather_kernel()(a, b, src, dst)
        em = _edge_math(g1, g2, ew3d, r(blk["msg_g"]), r(blk["msg_beta"]))
        msum = _sc_scatter_kernel()(em, dst3d, zeros_cw)
        x = _update(x, msum, cnts, blk["upd_W"], r(blk["upd_b"]),
                    r(blk["upd_g"]), r(blk["upd_beta"]),
                    r(blk["gate_W"][0]), blk["gate_b"].reshape(1, 1))

    return _out_proj(x, p["out_W"], r(p["out_b"]))
